# baseline (speedup 1.0000x reference)
import jax
import jax.numpy as jnp
from jax import lax
from jax.experimental import pallas as pl
from jax.experimental.pallas import tpu as pltpu
from jax.experimental.pallas import tpu_sc as plsc

N = 50000
D = 64
E = 800000
HALF = N // 2
QTR = N // 4
NS = 16
NC = 2
K = 224
CPT = 224
EPT = CPT * K
EPAD = NS * EPT
DUMMY = QTR
ACCR = 12512
DCH = 20
DROWS = QTR // DCH
SEG = EPT
NSEG = NC * 2 * NS
STG = 1136
FL = 4 * K


def _part_body(ei, w, colp, wp, locp, counts, rowb, colb, wb, st0, st1, cntb, esem):
    c = lax.axis_index("c")
    s = lax.axis_index("s")
    base_row = c * HALF
    ebase = s * EPT
    segid = [(c * 2 + r) * NS + s for r in range(2)]
    seg = [sid * SEG for sid in segid]
    stc = [st0[0], st1[0]]
    stw = [st0[1], st1[1]]
    stl = [st0[2], st1[2]]

    def wait_edges(p):
        pltpu.make_async_copy(ei.at[0, pl.ds(0, K)], rowb[p], esem[p]).wait()
        pltpu.make_async_copy(ei.at[1, pl.ds(0, K)], colb[p], esem[p]).wait()
        pltpu.make_async_copy(w.at[pl.ds(0, K)], wb[p], esem[p]).wait()

    def issue_edges(k, p):
        b0 = ebase + k * K
        pltpu.async_copy(ei.at[0, pl.ds(b0, K)], rowb[p], esem[p])
        pltpu.async_copy(ei.at[1, pl.ds(b0, K)], colb[p], esem[p])
        pltpu.async_copy(w.at[pl.ds(b0, K)], wb[p], esem[p])

    pltpu.sync_copy(ei.at[0, pl.ds(ebase, K)], rowb[0])
    pltpu.sync_copy(ei.at[1, pl.ds(ebase, K)], colb[0])
    pltpu.sync_copy(w.at[pl.ds(ebase, K)], wb[0])
    issue_edges(1, 1)

    def flush(r, sp, wof):
        full = sp >= FL

        @pl.when(full)
        def _():
            fo = pl.multiple_of(seg[r] + wof, 8)
            pltpu.sync_copy(stc[r].at[pl.ds(0, FL)], colp.at[pl.ds(fo, FL)])
            pltpu.sync_copy(stw[r].at[pl.ds(0, FL)], wp.at[pl.ds(fo, FL)])
            pltpu.sync_copy(stl[r].at[pl.ds(0, FL)], locp.at[pl.ds(fo, FL)])
            nmv = (sp - FL + 15) // 16

            def mv(i, u):
                stc[r][pl.ds(i * 16, 16)] = stc[r][pl.ds(FL + i * 16, 16)]
                stw[r][pl.ds(i * 16, 16)] = stw[r][pl.ds(FL + i * 16, 16)]
                stl[r][pl.ds(i * 16, 16)] = stl[r][pl.ds(FL + i * 16, 16)]
                return u

            lax.fori_loop(0, nmv, mv, 0)

        return jnp.where(full, sp - FL, sp), jnp.where(full, wof + FL, wof)

    def pair(j, carry):
        sp0, wof0, sp1, wof1 = carry
        for p in range(2):
            k = j * 2 + p

            @pl.when(k >= 1)
            def _():
                wait_edges(p)

            for g in range(K // 16):
                rv = rowb[p][pl.ds(g * 16, 16)]
                cv = colb[p][pl.ds(g * 16, 16)]
                wv = wb[p][pl.ds(g * 16, 16)]
                loc = rv - base_row
                in0 = (loc >= 0) & (loc < QTR)
                in1 = (loc >= QTR) & (loc < HALF)
                plsc.store_compressed(stc[0].at[pl.ds(sp0, 16)], cv, mask=in0)
                plsc.store_compressed(stw[0].at[pl.ds(sp0, 16)], wv, mask=in0)
                plsc.store_compressed(stl[0].at[pl.ds(sp0, 16)], loc, mask=in0)
                sp0 = sp0 + jnp.sum(in0.astype(jnp.int32))
                plsc.store_compressed(stc[1].at[pl.ds(sp1, 16)], cv, mask=in1)
                plsc.store_compressed(stw[1].at[pl.ds(sp1, 16)], wv, mask=in1)
                plsc.store_compressed(stl[1].at[pl.ds(sp1, 16)], loc - QTR, mask=in1)
                sp1 = sp1 + jnp.sum(in1.astype(jnp.int32))

            @pl.when(k + 2 < CPT)
            def _():
                issue_edges(k + 2, p)

            sp0, wof0 = flush(0, sp0, wof0)
            sp1, wof1 = flush(1, sp1, wof1)
        return sp0, wof0, sp1, wof1

    z = jnp.int32(0)
    sp0, wof0, sp1, wof1 = lax.fori_loop(0, CPT // 2, pair, (z, z, z, z))

    zc = jnp.zeros((16,), jnp.int32)
    zw = jnp.zeros((16,), jnp.float32)
    dl = jnp.full((16,), DUMMY, jnp.int32)

    for r, sp, wof in ((0, sp0, wof0), (1, sp1, wof1)):
        cnt = wof + sp
        target = jnp.maximum(FL, ((cnt + FL - 1) // FL) * FL)
        npad16 = (target - cnt + 15) // 16

        def padb(i, u, r=r, sp=sp):
            stc[r][pl.ds(sp + i * 16, 16)] = zc
            stw[r][pl.ds(sp + i * 16, 16)] = zw
            stl[r][pl.ds(sp + i * 16, 16)] = dl
            return u

        lax.fori_loop(0, npad16, padb, 0)

        @pl.when(target - wof > 0)
        def _(r=r, wof=wof):
            fo = pl.multiple_of(seg[r] + wof, 8)
            pltpu.sync_copy(stc[r].at[pl.ds(0, FL)], colp.at[pl.ds(fo, FL)])
            pltpu.sync_copy(stw[r].at[pl.ds(0, FL)], wp.at[pl.ds(fo, FL)])
            pltpu.sync_copy(stl[r].at[pl.ds(0, FL)], locp.at[pl.ds(fo, FL)])

        cntb[pl.ds(0, 16)] = jnp.zeros((16,), jnp.int32) + target
        pltpu.sync_copy(cntb, counts.at[pl.ds(segid[r] * 16, 16)])


def _layer_body(h, colp, wp, locp, counts, zb, out, acc, colb, wb, locb, msg, sidxa, sidxb, cntv, gsem, ssem, esem):
    c = lax.axis_index("c")
    s = lax.axis_index("s")

    def wait_msg(sem, buf):
        pltpu.make_async_copy(h.at[pl.ds(0, K)], buf, sem).wait()

    def wait_edges(b):
        pltpu.make_async_copy(colp.at[pl.ds(0, K)], colb[b], esem[b]).wait()
        pltpu.make_async_copy(wp.at[pl.ds(0, K)], wb[b], esem[b]).wait()
        pltpu.make_async_copy(locp.at[pl.ds(0, K)], locb[b], esem[b]).wait()

    for rg in range(2):
        base_q = c * HALF + rg * QTR
        segid = (c * 2 + rg) * NS + s
        seg = segid * SEG

        for jj in range(3):
            pltpu.sync_copy(zb, acc.at[pl.ds((s + jj * NS) * 256, 256)])

        @pl.when(s == 0)
        def _():
            pltpu.sync_copy(zb.at[pl.ds(0, 224)], acc.at[pl.ds(48 * 256, 224)])

        plsc.subcore_barrier()

        pltpu.sync_copy(counts.at[pl.ds(segid * 16, 16)], cntv)
        cnt = jnp.max(cntv[pl.ds(0, 16)])
        nb = cnt // K

        def load_edges_sync(k, b, seg=seg):
            koff = seg + k * K
            pltpu.sync_copy(colp.at[pl.ds(koff, K)], colb[b])
            pltpu.sync_copy(wp.at[pl.ds(koff, K)], wb[b])
            pltpu.sync_copy(locp.at[pl.ds(koff, K)], locb[b])

        def issue_edges(k, b, seg=seg):
            koff = seg + k * K
            pltpu.async_copy(colp.at[pl.ds(koff, K)], colb[b], esem[b])
            pltpu.async_copy(wp.at[pl.ds(koff, K)], wb[b], esem[b])
            pltpu.async_copy(locp.at[pl.ds(koff, K)], locb[b], esem[b])

        load_edges_sync(0, 0)
        load_edges_sync(1, 1)
        pltpu.async_copy(h.at[colb[0]], msg[0], gsem[0])
        pltpu.async_copy(h.at[colb[1]], msg[1], gsem[1])
        issue_edges(2, 2)
        issue_edges(3, 3)

        def quad(j, carry, nb=nb):
            for b in range(4):
                k = j * 4 + b
                b2 = (b + 2) % 4

                @pl.when(k + 2 < nb)
                def _(b2=b2, k=k):
                    @pl.when(k >= 2)
                    def _():
                        wait_msg(ssem[b2], msg[b2])

                    wait_edges(b2)
                    pltpu.async_copy(h.at[colb[b2]], msg[b2], gsem[b2])

                wait_msg(gsem[b], msg[b])

                @plsc.parallel_loop(0, K, 1, unroll=8)
                def _(e0, b=b):
                    wv = plsc.load_gather(wb[b], [jnp.full((16,), e0, jnp.int32)])
                    for d in range(D // 16):
                        msg[b][e0, pl.ds(d * 16, 16)] = msg[b][e0, pl.ds(d * 16, 16)] * wv

                for g in range(8):
                    sidxa[b][0, pl.ds(g * 16, 16)] = locb[b][pl.ds(g * 16, 16)]
                for g in range(6):
                    sidxb[b][0, pl.ds(g * 16, 16)] = locb[b][pl.ds(128 + g * 16, 16)]
                pltpu.async_copy(msg[b].at[pl.ds(0, 128)], acc.at[sidxa[b].at[0]], ssem[b], add=True)
                pltpu.async_copy(msg[b].at[pl.ds(128, 96)], acc.at[sidxb[b].at[0]], ssem[b], add=True)

                @pl.when(k + 4 < nb)
                def _(k=k, b=b):
                    issue_edges(k + 4, b)

            return carry

        lax.fori_loop(0, nb // 4, quad, 0)

        for b in range(4):
            wait_msg(ssem[b], msg[b])

        plsc.subcore_barrier()

        for jj in range(2):
            j = s + jj * NS

            @pl.when(j < DCH)
            def _(j=j, base_q=base_q):
                r0 = j * DROWS
                pltpu.sync_copy(acc.at[pl.ds(r0, DROWS)], out.at[pl.ds(base_q + r0, DROWS)])

        plsc.subcore_barrier()


def _prep_body(emb, xi, out, xb, rows, sem):
    c = lax.axis_index("c")
    s = lax.axis_index("s")
    wid = s * NC + c
    for jj in range(4):
        j = wid + jj * NS * NC

        @pl.when(j < 125)
        def _(j=j):
            r0 = j * 400
            pltpu.sync_copy(xi.at[pl.ds(r0, 400)], xb)
            pltpu.async_copy(emb.at[xb], rows, sem).wait()
            pltpu.sync_copy(rows, out.at[pl.ds(r0, 400)])


def _combine_body(a, b, c, d, o):
    o[...] = (a[...] + b[...] + c[...] + d[...]) * 0.25


def _build():
    mesh = plsc.VectorSubcoreMesh(core_axis_name="c", subcore_axis_name="s")
    sc_params = pltpu.CompilerParams(
        use_tc_tiling_on_sc=False, needs_layout_passes=False
    )

    part = pl.kernel(
        _part_body,
        out_type=(
            jax.ShapeDtypeStruct((NSEG * SEG,), jnp.int32),
            jax.ShapeDtypeStruct((NSEG * SEG,), jnp.float32),
            jax.ShapeDtypeStruct((NSEG * SEG,), jnp.int32),
            jax.ShapeDtypeStruct((NSEG * 16,), jnp.int32),
        ),
        mesh=mesh,
        compiler_params=sc_params,
        scratch_types=[
            [pltpu.VMEM((K,), jnp.int32)] * 2,
            [pltpu.VMEM((K,), jnp.int32)] * 2,
            [pltpu.VMEM((K,), jnp.float32)] * 2,
            [
                pltpu.VMEM((STG,), jnp.int32),
                pltpu.VMEM((STG,), jnp.float32),
                pltpu.VMEM((STG,), jnp.int32),
            ],
            [
                pltpu.VMEM((STG,), jnp.int32),
                pltpu.VMEM((STG,), jnp.float32),
                pltpu.VMEM((STG,), jnp.int32),
            ],
            pltpu.VMEM((16,), jnp.int32),
            [pltpu.SemaphoreType.DMA] * 2,
        ],
    )

    layer = pl.kernel(
        _layer_body,
        out_type=jax.ShapeDtypeStruct((N, D), jnp.float32),
        mesh=mesh,
        compiler_params=sc_params,
        scratch_types=[
            pltpu.VMEM_SHARED((ACCR, D), jnp.float32),
            [pltpu.VMEM((K,), jnp.int32)] * 4,
            [pltpu.VMEM((K,), jnp.float32)] * 4,
            [pltpu.VMEM((K,), jnp.int32)] * 4,
            [pltpu.VMEM((K, D), jnp.float32)] * 4,
            [pltpu.VMEM((1, 128), jnp.int32)] * 4,
            [pltpu.VMEM((1, 96), jnp.int32)] * 4,
            pltpu.VMEM((16,), jnp.int32),
            [pltpu.SemaphoreType.DMA] * 4,
            [pltpu.SemaphoreType.DMA] * 4,
            [pltpu.SemaphoreType.DMA] * 4,
        ],
    )

    prep = pl.kernel(
        _prep_body,
        out_type=jax.ShapeDtypeStruct((N, D), jnp.float32),
        mesh=mesh,
        compiler_params=sc_params,
        scratch_types=[
            pltpu.VMEM((400,), jnp.int32),
            pltpu.VMEM((400, D), jnp.float32),
            pltpu.SemaphoreType.DMA,
        ],
    )

    combine = pl.pallas_call(
        _combine_body,
        out_shape=jax.ShapeDtypeStruct((N, D), jnp.float32),
        grid=(125,),
        in_specs=[pl.BlockSpec((400, D), lambda i: (i, 0))] * 4,
        out_specs=pl.BlockSpec((400, D), lambda i: (i, 0)),
    )
    return prep, part, layer, combine


@jax.jit
def kernel(x, edge_index, edge_weight, embed_weight):
    prep, part, layer, combine = _build()
    pad = EPAD - E
    ei = jnp.pad(edge_index.astype(jnp.int32), ((0, 0), (0, pad)))
    ww = jnp.pad(edge_weight, (0, pad))
    zb = jnp.zeros((256, D), jnp.float32)
    h0 = prep(embed_weight, x.astype(jnp.int32))
    colp, wp, locp, counts = part(ei, ww)
    h1 = layer(h0, colp, wp, locp, counts, zb)
    h2 = layer(h1, colp, wp, locp, counts, zb)
    h3 = layer(h2, colp, wp, locp, counts, zb)
    return combine(h0, h1, h2, h3)


# speedup vs baseline: 1.8761x; 1.8761x over previous
import jax
import jax.numpy as jnp
from jax import lax
from jax.experimental import pallas as pl
from jax.experimental.pallas import tpu as pltpu
from jax.experimental.pallas import tpu_sc as plsc

N = 50000
D = 64
E = 800000
HALF = N // 2
NS = 16
NC = 2
K = 224
CPT = 224
EPT = CPT * K
EPAD = NS * EPT
DUMMY = HALF
ACCR = 25008
ZCH = 97
DCH = 40
DROWS = HALF // DCH
SEG = EPT
NSEG = NC * NS
STG = 912
FL = 2 * K


def _part_body(ei, w, colp, wp, locp, counts, rowb, colb, wb, stc, stw, stl, cntb, esem):
    c = lax.axis_index("c")
    s = lax.axis_index("s")
    base_row = c * HALF
    wid = c * NS + s
    seg = wid * SEG
    ebase = s * EPT

    def wait_edges(p):
        pltpu.make_async_copy(ei.at[0, pl.ds(0, K)], rowb[p], esem[p]).wait()
        pltpu.make_async_copy(ei.at[1, pl.ds(0, K)], colb[p], esem[p]).wait()
        pltpu.make_async_copy(w.at[pl.ds(0, K)], wb[p], esem[p]).wait()

    def issue_edges(k, p):
        b0 = ebase + k * K
        pltpu.async_copy(ei.at[0, pl.ds(b0, K)], rowb[p], esem[p])
        pltpu.async_copy(ei.at[1, pl.ds(b0, K)], colb[p], esem[p])
        pltpu.async_copy(w.at[pl.ds(b0, K)], wb[p], esem[p])

    pltpu.sync_copy(ei.at[0, pl.ds(ebase, K)], rowb[0])
    pltpu.sync_copy(ei.at[1, pl.ds(ebase, K)], colb[0])
    pltpu.sync_copy(w.at[pl.ds(ebase, K)], wb[0])
    issue_edges(1, 1)

    def pair(j, carry):
        sp, wof = carry
        for p in range(2):
            k = j * 2 + p

            @pl.when(k >= 1)
            def _():
                wait_edges(p)

            for g in range(K // 16):
                rv = rowb[p][pl.ds(g * 16, 16)]
                cv = colb[p][pl.ds(g * 16, 16)]
                wv = wb[p][pl.ds(g * 16, 16)]
                loc = rv - base_row
                inb = (loc >= 0) & (loc < HALF)
                plsc.store_compressed(stc.at[pl.ds(sp, 16)], cv, mask=inb)
                plsc.store_compressed(stw.at[pl.ds(sp, 16)], wv, mask=inb)
                plsc.store_compressed(stl.at[pl.ds(sp, 16)], loc, mask=inb)
                sp = sp + jnp.sum(inb.astype(jnp.int32))

            @pl.when(k + 2 < CPT)
            def _():
                issue_edges(k + 2, p)

            full = sp >= FL

            @pl.when(full)
            def _():
                fo = pl.multiple_of(seg + wof, 8)
                pltpu.sync_copy(stc.at[pl.ds(0, FL)], colp.at[pl.ds(fo, FL)])
                pltpu.sync_copy(stw.at[pl.ds(0, FL)], wp.at[pl.ds(fo, FL)])
                pltpu.sync_copy(stl.at[pl.ds(0, FL)], locp.at[pl.ds(fo, FL)])
                nmv = (sp - FL + 15) // 16

                def mv(i, u):
                    stc[pl.ds(i * 16, 16)] = stc[pl.ds(FL + i * 16, 16)]
                    stw[pl.ds(i * 16, 16)] = stw[pl.ds(FL + i * 16, 16)]
                    stl[pl.ds(i * 16, 16)] = stl[pl.ds(FL + i * 16, 16)]
                    return u

                lax.fori_loop(0, nmv, mv, 0)

            sp = jnp.where(full, sp - FL, sp)
            wof = jnp.where(full, wof + FL, wof)
        return sp, wof

    sp, wof = lax.fori_loop(0, CPT // 2, pair, (jnp.int32(0), jnp.int32(0)))

    cnt = wof + sp
    target = jnp.maximum(FL, ((cnt + FL - 1) // FL) * FL)
    npad16 = (target - cnt + 15) // 16
    zc = jnp.zeros((16,), jnp.int32)
    zw = jnp.zeros((16,), jnp.float32)
    dl = jnp.full((16,), DUMMY, jnp.int32)

    def padb(i, u):
        stc[pl.ds(sp + i * 16, 16)] = zc
        stw[pl.ds(sp + i * 16, 16)] = zw
        stl[pl.ds(sp + i * 16, 16)] = dl
        return u

    lax.fori_loop(0, npad16, padb, 0)

    @pl.when(target - wof > 0)
    def _():
        fo = pl.multiple_of(seg + wof, 8)
        pltpu.sync_copy(stc.at[pl.ds(0, FL)], colp.at[pl.ds(fo, FL)])
        pltpu.sync_copy(stw.at[pl.ds(0, FL)], wp.at[pl.ds(fo, FL)])
        pltpu.sync_copy(stl.at[pl.ds(0, FL)], locp.at[pl.ds(fo, FL)])

    cntb[pl.ds(0, 16)] = jnp.zeros((16,), jnp.int32) + target
    pltpu.sync_copy(cntb, counts.at[pl.ds(wid * 16, 16)])


def _layer_body(h, colp, wp, locp, counts, zb, out, acc, colb, wb, locb, msg, idxa, idxb, cntv, gsem, ssem, esem):
    c = lax.axis_index("c")
    s = lax.axis_index("s")
    base_row = c * HALF
    wid = c * NS + s
    seg = wid * SEG

    for jj in range(7):
        j = s + jj * NS

        @pl.when(j < ZCH)
        def _():
            pltpu.sync_copy(zb, acc.at[pl.ds(j * 256, 256)])

    @pl.when(s == 0)
    def _():
        pltpu.sync_copy(zb.at[pl.ds(0, 176)], acc.at[pl.ds(ZCH * 256, 176)])

    plsc.subcore_barrier()

    pltpu.sync_copy(counts.at[pl.ds(wid * 16, 16)], cntv)
    cnt = jnp.max(cntv[pl.ds(0, 16)])
    nb = cnt // K

    def wait_msg(sem, buf):
        pltpu.make_async_copy(h.at[pl.ds(0, K)], buf, sem).wait()

    def wait_edges(p):
        pltpu.make_async_copy(colp.at[pl.ds(0, K)], colb[p], esem[p]).wait()
        pltpu.make_async_copy(wp.at[pl.ds(0, K)], wb[p], esem[p]).wait()
        pltpu.make_async_copy(locp.at[pl.ds(0, K)], locb[p], esem[p]).wait()

    def load_edges_sync(k, p):
        koff = seg + k * K
        pltpu.sync_copy(colp.at[pl.ds(koff, K)], colb[p])
        pltpu.sync_copy(wp.at[pl.ds(koff, K)], wb[p])
        pltpu.sync_copy(locp.at[pl.ds(koff, K)], locb[p])

    def issue_edges(k, p):
        koff = seg + k * K
        pltpu.async_copy(colp.at[pl.ds(koff, K)], colb[p], esem[p])
        pltpu.async_copy(wp.at[pl.ds(koff, K)], wb[p], esem[p])
        pltpu.async_copy(locp.at[pl.ds(koff, K)], locb[p], esem[p])

    load_edges_sync(0, 0)
    load_edges_sync(1, 1)
    pltpu.async_copy(h.at[colb[0]], msg[0], gsem[0])

    def pair(j, carry):
        for p in range(2):
            k = j * 2 + p
            q = 1 - p

            @pl.when(k + 1 < nb)
            def _():
                @pl.when(k >= 1)
                def _():
                    wait_msg(ssem[q], msg[q])
                    wait_edges(q)

                pltpu.async_copy(h.at[colb[q]], msg[q], gsem[q])

            wait_msg(gsem[p], msg[p])

            @plsc.parallel_loop(0, K, 1, unroll=8)
            def _(e0):
                wv = plsc.load_gather(wb[p], [jnp.full((16,), e0, jnp.int32)])
                for d in range(D // 16):
                    msg[p][e0, pl.ds(d * 16, 16)] = msg[p][e0, pl.ds(d * 16, 16)] * wv

            for g in range(8):
                idxa[p][0, pl.ds(g * 16, 16)] = locb[p][pl.ds(g * 16, 16)]
            for g in range(6):
                idxb[p][0, pl.ds(g * 16, 16)] = locb[p][pl.ds(128 + g * 16, 16)]
            pltpu.async_copy(msg[p].at[pl.ds(0, 128)], acc.at[idxa[p].at[0]], ssem[p], add=True)
            pltpu.async_copy(msg[p].at[pl.ds(128, 96)], acc.at[idxb[p].at[0]], ssem[p], add=True)

            @pl.when(k + 2 < nb)
            def _():
                issue_edges(k + 2, p)

        return carry

    lax.fori_loop(0, nb // 2, pair, 0)

    wait_msg(ssem[0], msg[0])
    wait_msg(ssem[1], msg[1])

    plsc.subcore_barrier()

    for jj in range(3):
        j = s + jj * NS

        @pl.when(j < DCH)
        def _():
            r0 = j * DROWS
            pltpu.sync_copy(acc.at[pl.ds(r0, DROWS)], out.at[pl.ds(base_row + r0, DROWS)])


def _prep_body(emb, xi, out, xb, rows, sem):
    c = lax.axis_index("c")
    s = lax.axis_index("s")
    wid = s * NC + c
    for jj in range(4):
        j = wid + jj * NS * NC

        @pl.when(j < 125)
        def _():
            r0 = j * 400
            pltpu.sync_copy(xi.at[pl.ds(r0, 400)], xb)
            pltpu.async_copy(emb.at[xb], rows, sem).wait()
            pltpu.sync_copy(rows, out.at[pl.ds(r0, 400)])


def _combine_body(a, b, c, d, o):
    o[...] = (a[...] + b[...] + c[...] + d[...]) * 0.25


def _build():
    mesh = plsc.VectorSubcoreMesh(core_axis_name="c", subcore_axis_name="s")
    sc_params = pltpu.CompilerParams(
        use_tc_tiling_on_sc=False, needs_layout_passes=False
    )

    part = pl.kernel(
        _part_body,
        out_type=(
            jax.ShapeDtypeStruct((NSEG * SEG,), jnp.int32),
            jax.ShapeDtypeStruct((NSEG * SEG,), jnp.float32),
            jax.ShapeDtypeStruct((NSEG * SEG,), jnp.int32),
            jax.ShapeDtypeStruct((NSEG * 16,), jnp.int32),
        ),
        mesh=mesh,
        compiler_params=sc_params,
        scratch_types=[
            [pltpu.VMEM((K,), jnp.int32)] * 2,
            [pltpu.VMEM((K,), jnp.int32)] * 2,
            [pltpu.VMEM((K,), jnp.float32)] * 2,
            pltpu.VMEM((STG,), jnp.int32),
            pltpu.VMEM((STG,), jnp.float32),
            pltpu.VMEM((STG,), jnp.int32),
            pltpu.VMEM((16,), jnp.int32),
            [pltpu.SemaphoreType.DMA] * 2,
        ],
    )

    layer = pl.kernel(
        _layer_body,
        out_type=jax.ShapeDtypeStruct((N, D), jnp.float32),
        mesh=mesh,
        compiler_params=sc_params,
        scratch_types=[
            pltpu.VMEM_SHARED((ACCR, D), jnp.float32),
            [pltpu.VMEM((K,), jnp.int32)] * 2,
            [pltpu.VMEM((K,), jnp.float32)] * 2,
            [pltpu.VMEM((K,), jnp.int32)] * 2,
            [pltpu.VMEM((K, D), jnp.float32)] * 2,
            [pltpu.VMEM((1, 128), jnp.int32)] * 2,
            [pltpu.VMEM((1, 96), jnp.int32)] * 2,
            pltpu.VMEM((16,), jnp.int32),
            [pltpu.SemaphoreType.DMA] * 2,
            [pltpu.SemaphoreType.DMA] * 2,
            [pltpu.SemaphoreType.DMA] * 2,
        ],
    )

    prep = pl.kernel(
        _prep_body,
        out_type=jax.ShapeDtypeStruct((N, D), jnp.float32),
        mesh=mesh,
        compiler_params=sc_params,
        scratch_types=[
            pltpu.VMEM((400,), jnp.int32),
            pltpu.VMEM((400, D), jnp.float32),
            pltpu.SemaphoreType.DMA,
        ],
    )

    combine = pl.pallas_call(
        _combine_body,
        out_shape=jax.ShapeDtypeStruct((N, D), jnp.float32),
        grid=(125,),
        in_specs=[pl.BlockSpec((400, D), lambda i: (i, 0))] * 4,
        out_specs=pl.BlockSpec((400, D), lambda i: (i, 0)),
    )
    return prep, part, layer, combine


@jax.jit
def kernel(x, edge_index, edge_weight, embed_weight):
    prep, part, layer, combine = _build()
    pad = EPAD - E
    ei = jnp.pad(edge_index.astype(jnp.int32), ((0, 0), (0, pad)))
    ww = jnp.pad(edge_weight, (0, pad))
    zb = jnp.zeros((256, D), jnp.float32)
    h0 = prep(embed_weight, x.astype(jnp.int32))
    colp, wp, locp, counts = part(ei, ww)
    h1 = layer(h0, colp, wp, locp, counts, zb)
    h2 = layer(h1, colp, wp, locp, counts, zb)
    h3 = layer(h2, colp, wp, locp, counts, zb)
    return combine(h0, h1, h2, h3)
